# baseline (speedup 1.0000x reference)
import jax
import jax.numpy as jnp
import numpy as np
from jax.experimental import pallas as pl
from jax.experimental.pallas import tpu as pltpu

_ROWS = 8
_W = 512
_TINY = np.float32(np.finfo(np.float32).tiny)
_SCALE = np.float32(1.0) - _TINY


def _threefry_bits(j):
    ks0 = jnp.uint32(0)
    ks1 = jnp.uint32(42)
    ks2 = jnp.uint32(0 ^ 42 ^ 0x1BD11BDA)

    def rotl(x, d):
        return (x << jnp.uint32(d)) | (x >> jnp.uint32(32 - d))

    def four_rounds(x0, x1, rots):
        for r in rots:
            x0 = x0 + x1
            x1 = rotl(x1, r)
            x1 = x0 ^ x1
        return x0, x1

    r1 = (13, 15, 26, 6)
    r2 = (17, 29, 16, 24)
    x0 = jnp.zeros_like(j) + ks0
    x1 = j + ks1
    x0, x1 = four_rounds(x0, x1, r1)
    x0 = x0 + ks1
    x1 = x1 + ks2 + jnp.uint32(1)
    x0, x1 = four_rounds(x0, x1, r2)
    x0 = x0 + ks2
    x1 = x1 + ks0 + jnp.uint32(2)
    x0, x1 = four_rounds(x0, x1, r1)
    x0 = x0 + ks0
    x1 = x1 + ks1 + jnp.uint32(3)
    x0, x1 = four_rounds(x0, x1, r2)
    x0 = x0 + ks1
    x1 = x1 + ks2 + jnp.uint32(4)
    x0, x1 = four_rounds(x0, x1, r1)
    x0 = x0 + ks2
    x1 = x1 + ks0 + jnp.uint32(5)
    return x0 ^ x1


def _gumbel(j):
    bits = _threefry_bits(j)
    fbits = (bits >> jnp.uint32(9)) | jnp.uint32(0x3F800000)
    f = jax.lax.bitcast_convert_type(fbits, jnp.float32) - jnp.float32(1.0)
    u = jnp.maximum(_TINY, f * _SCALE + _TINY)
    return -jnp.log(-jnp.log(u))


def _make_kernel(vocab, n_chunks):
    last = n_chunks - 1

    def body(x_ref, o_ref, best_ref, bidx_ref):
        i = pl.program_id(0)
        t = pl.program_id(1)

        @pl.when(t == 0)
        def _init():
            best_ref[...] = jnp.full((_ROWS, _W), -jnp.inf, jnp.float32)
            bidx_ref[...] = jnp.zeros((_ROWS, _W), jnp.int32)

        sub = jax.lax.broadcasted_iota(jnp.int32, (_ROWS, _W), 0)
        lane = jax.lax.broadcasted_iota(jnp.int32, (_ROWS, _W), 1)
        col = t * _W + lane
        row = i * _ROWS + sub
        j = (row * vocab + col).astype(jnp.uint32)

        z = _gumbel(j) + x_ref[...]
        z = jnp.where(col < vocab, z, -jnp.inf)

        best = best_ref[...]
        upd = z > best
        best_ref[...] = jnp.where(upd, z, best)
        bidx_ref[...] = jnp.where(upd, col, bidx_ref[...])

        @pl.when(t == last)
        def _finish():
            b = best_ref[...]
            ix = bidx_ref[...]
            gmax = jnp.max(b, axis=1, keepdims=True)
            tok = jnp.min(jnp.where(b == gmax, ix, vocab), axis=1, keepdims=True)
            o_ref[...] = tok

    return body


def kernel(logits):
    b, l, vocab = logits.shape
    rows = b * l
    x = logits.reshape(rows, vocab)
    n_chunks = pl.cdiv(vocab, _W)
    grid = (rows // _ROWS, n_chunks)
    out = pl.pallas_call(
        _make_kernel(vocab, n_chunks),
        grid=grid,
        in_specs=[pl.BlockSpec((_ROWS, _W), lambda i, t: (i, t))],
        out_specs=pl.BlockSpec((_ROWS, 1), lambda i, t: (i, 0)),
        out_shape=jax.ShapeDtypeStruct((rows, 1), jnp.int32),
        scratch_shapes=[
            pltpu.VMEM((_ROWS, _W), jnp.float32),
            pltpu.VMEM((_ROWS, _W), jnp.int32),
        ],
    )(x)
    return out.reshape(b, l)


# speedup vs baseline: 3.1204x; 3.1204x over previous
import jax
import jax.numpy as jnp
import numpy as np
from jax.experimental import pallas as pl
from jax.experimental.pallas import tpu as pltpu

_ROWS = 8
_W = 2048
_TINY = np.float32(np.finfo(np.float32).tiny)


def _threefry_bits(j):
    ks1 = jnp.uint32(42)
    ks2 = jnp.uint32(42 ^ 0x1BD11BDA)

    def rotl(x, d):
        return (x << jnp.uint32(d)) | (x >> jnp.uint32(32 - d))

    def four_rounds(x0, x1, rots):
        for r in rots:
            x0 = x0 + x1
            x1 = rotl(x1, r)
            x1 = x0 ^ x1
        return x0, x1

    r1 = (13, 15, 26, 6)
    r2 = (17, 29, 16, 24)
    a = j + ks1
    x0 = a
    x1 = rotl(a, 13) ^ a
    x0, x1 = four_rounds(x0, x1, (15, 26, 6))
    x0 = x0 + ks1
    x1 = x1 + (ks2 + jnp.uint32(1))
    x0, x1 = four_rounds(x0, x1, r2)
    x0 = x0 + ks2
    x1 = x1 + jnp.uint32(2)
    x0, x1 = four_rounds(x0, x1, r1)
    x0 = x0
    x1 = x1 + (ks1 + jnp.uint32(3))
    x0, x1 = four_rounds(x0, x1, r2)
    x0 = x0 + ks1
    x1 = x1 + (ks2 + jnp.uint32(4))
    x0, x1 = four_rounds(x0, x1, r1)
    x0 = x0 + ks2
    x1 = x1 + jnp.uint32(5)
    return x0 ^ x1


def _gumbel(j):
    bits = _threefry_bits(j)
    fbits = (bits >> jnp.uint32(9)) | jnp.uint32(0x3F800000)
    f = jax.lax.bitcast_convert_type(fbits, jnp.float32) - jnp.float32(1.0)
    u = jnp.maximum(f, _TINY)
    return -jnp.log(-jnp.log(u))


def _make_kernel(vocab, n_chunks, w):
    last = n_chunks - 1
    tail = vocab - last * w
    shift = int(np.log2(w))

    def body(x_ref, o_ref, best_ref, btid_ref, jb_ref):
        i = pl.program_id(0)
        t = pl.program_id(1)

        @pl.when(t == 0)
        def _init():
            sub = jax.lax.broadcasted_iota(jnp.int32, (_ROWS, w), 0)
            lane = jax.lax.broadcasted_iota(jnp.int32, (_ROWS, w), 1)
            row = i * _ROWS + sub
            jb_ref[...] = (row * vocab + lane).astype(jnp.uint32)
            best_ref[...] = jnp.full((_ROWS, w), -jnp.inf, jnp.float32)
            btid_ref[...] = jnp.zeros((_ROWS, w), jnp.int32)

        j = jb_ref[...] + (t * w).astype(jnp.uint32)
        z = _gumbel(j) + x_ref[...]

        def update(zv):
            prev = best_ref[...]
            b = jnp.maximum(prev, zv)
            m = b != prev
            best_ref[...] = b
            btid_ref[...] = jnp.where(m, t, btid_ref[...])

        @pl.when(t != last)
        def _mid():
            update(z)

        @pl.when(t == last)
        def _fin():
            lane = jax.lax.broadcasted_iota(jnp.int32, (_ROWS, w), 1)
            update(jnp.where(lane < tail, z, -jnp.inf))
            b = best_ref[...]
            col = (btid_ref[...] << shift) + lane
            gmax = jnp.max(b, axis=1, keepdims=True)
            tok = jnp.min(jnp.where(b == gmax, col, vocab), axis=1,
                          keepdims=True)
            o_ref[...] = tok

    return body


def kernel(logits):
    b, l, vocab = logits.shape
    rows = b * l
    x = logits.reshape(rows, vocab)
    n_chunks = pl.cdiv(vocab, _W)
    grid = (rows // _ROWS, n_chunks)
    out = pl.pallas_call(
        _make_kernel(vocab, n_chunks, _W),
        grid=grid,
        in_specs=[pl.BlockSpec((_ROWS, _W), lambda i, t: (i, t))],
        out_specs=pl.BlockSpec((_ROWS, 1), lambda i, t: (i, 0)),
        out_shape=jax.ShapeDtypeStruct((rows, 1), jnp.int32),
        scratch_shapes=[
            pltpu.VMEM((_ROWS, _W), jnp.float32),
            pltpu.VMEM((_ROWS, _W), jnp.int32),
            pltpu.VMEM((_ROWS, _W), jnp.uint32),
        ],
    )(x)
    return out.reshape(b, l)
